# baseline (speedup 1.0000x reference)
import functools

import jax
import jax.numpy as jnp
from jax import lax
from jax.experimental import pallas as pl
from jax.experimental.pallas import tpu as pltpu
from jax.experimental.pallas import tpu_sc as plsc

B = 4096
H = 50
EMB = 32
BLK = 128
NC = 2
NS = 16
NW = NC * NS
BPW = B // NW
LANES = 16
ROW_VREGS = EMB // LANES


def _sc_gather_pool(vid_g, wvt, region_g, cid_g, vemb4, remb4, cemb4):
  mesh = plsc.VectorSubcoreMesh(core_axis_name="c", subcore_axis_name="s")

  @functools.partial(
      pl.kernel,
      mesh=mesh,
      out_type=(
          jax.ShapeDtypeStruct((B, BLK), jnp.float32),
          jax.ShapeDtypeStruct((B, EMB), jnp.float32),
          jax.ShapeDtypeStruct((B, BLK), jnp.float32),
          jax.ShapeDtypeStruct((B, BLK), jnp.float32),
      ),
      scratch_types=[
          pltpu.VMEM((BPW,), jnp.int32),
          pltpu.VMEM((BPW,), jnp.int32),
          pltpu.VMEM((BPW,), jnp.int32),
          pltpu.VMEM((H, BPW), jnp.int32),
          pltpu.VMEM((BPW,), jnp.int32),
          pltpu.VMEM((BPW,), jnp.int32),
          pltpu.VMEM((BPW, BLK), jnp.float32),
          pltpu.VMEM((BPW, BLK), jnp.float32),
          pltpu.VMEM((BPW, BLK), jnp.float32),
          pltpu.VMEM((BPW, BLK), jnp.float32),
          pltpu.VMEM((BPW, BLK), jnp.float32),
          pltpu.VMEM((BPW, EMB), jnp.float32),
          pltpu.SemaphoreType.DMA,
          pltpu.SemaphoreType.DMA,
          pltpu.SemaphoreType.DMA,
          pltpu.SemaphoreType.DMA,
          pltpu.SemaphoreType.DMA,
      ],
  )
  def sc_kernel(vidg_hbm, wvt_hbm, regg_hbm, cidg_hbm,
                vemb_hbm, remb_hbm, cemb_hbm,
                out_v, out_p, out_r, out_c,
                vid_v, reg_v, cid_v, wvt_v, hg_a, hg_b,
                v_rows, r_rows, c_rows, buf_a, buf_b, acc,
                sem_a, sem_b, sem_v, sem_r, sem_c):
    wid = lax.axis_index("s") * NC + lax.axis_index("c")
    base = wid * BPW
    pltpu.sync_copy(vidg_hbm.at[pl.ds(base, BPW)], vid_v)
    pltpu.sync_copy(regg_hbm.at[pl.ds(base, BPW)], reg_v)
    pltpu.sync_copy(cidg_hbm.at[pl.ds(base, BPW)], cid_v)
    pltpu.sync_copy(wvt_hbm.at[:, pl.ds(base, BPW)], wvt_v)
    cp_v = pltpu.async_copy(vemb_hbm.at[vid_v], v_rows, sem_v)
    cp_r = pltpu.async_copy(remb_hbm.at[reg_v], r_rows, sem_r)
    cp_c = pltpu.async_copy(cemb_hbm.at[cid_v], c_rows, sem_c)

    def stage_blocks(h, hg):
      for q in range(BPW // LANES):
        hg[pl.ds(q * LANES, LANES)] = jnp.right_shift(
            wvt_v[h, pl.ds(q * LANES, LANES)], 2)

    def accumulate(buf, h):
      def add_chunk(q, carry):
        qb = pl.multiple_of(q * LANES, LANES)
        off_vec = jnp.bitwise_and(wvt_v[h, pl.ds(qb, LANES)], 3) * EMB
        for r in range(LANES):
          off = pl.multiple_of(off_vec[r], EMB)
          for j in range(ROW_VREGS):
            plsc.addupdate(acc.at[qb + r, pl.ds(j * LANES, LANES)],
                           buf[qb + r, pl.ds(off + j * LANES, LANES)])
        return carry

      lax.fori_loop(0, BPW // LANES, add_chunk, 0)

    def zero_body(b, carry):
      for j in range(ROW_VREGS):
        acc[b, pl.ds(j * LANES, LANES)] = jnp.zeros((LANES,), jnp.float32)
      return carry

    lax.fori_loop(0, BPW, zero_body, 0)
    stage_blocks(0, hg_a)
    pltpu.async_copy(vemb_hbm.at[hg_a], buf_a, sem_a)
    stage_blocks(1, hg_b)
    pltpu.async_copy(vemb_hbm.at[hg_b], buf_b, sem_b)

    def pair_body(t, carry):
      h = t * 2
      pltpu.make_async_copy(vemb_hbm.at[hg_a], buf_a, sem_a).wait()
      accumulate(buf_a, h)
      stage_blocks(h + 2, hg_a)
      pltpu.async_copy(vemb_hbm.at[hg_a], buf_a, sem_a)
      pltpu.make_async_copy(vemb_hbm.at[hg_b], buf_b, sem_b).wait()
      accumulate(buf_b, h + 1)
      stage_blocks(h + 3, hg_b)
      pltpu.async_copy(vemb_hbm.at[hg_b], buf_b, sem_b)
      return carry

    lax.fori_loop(0, H // 2 - 1, pair_body, 0)
    pltpu.make_async_copy(vemb_hbm.at[hg_a], buf_a, sem_a).wait()
    accumulate(buf_a, H - 2)
    pltpu.make_async_copy(vemb_hbm.at[hg_b], buf_b, sem_b).wait()
    accumulate(buf_b, H - 1)
    cp_v.wait()
    cp_r.wait()
    cp_c.wait()
    pltpu.sync_copy(v_rows, out_v.at[pl.ds(base, BPW)])
    pltpu.sync_copy(acc, out_p.at[pl.ds(base, BPW)])
    pltpu.sync_copy(r_rows, out_r.at[pl.ds(base, BPW)])
    pltpu.sync_copy(c_rows, out_c.at[pl.ds(base, BPW)])

  return sc_kernel(vid_g, wvt, region_g, cid_g, vemb4, remb4, cemb4)


def _select_sub(blocks_ref, sub_ref):
  blocks = blocks_ref[...]
  sub = sub_ref[...]
  out = jnp.zeros((blocks.shape[0], EMB), jnp.float32)
  for s in range(4):
    piece = blocks[:, s * EMB:(s + 1) * EMB]
    out = jnp.where(sub == s, piece, out)
  return out


def _mlp_body(v_ref, vs_ref, p_ref, r_ref, rs_ref, c_ref, cs_ref,
              w0, b0, w1, b1, w2, b2, wo, bo, out_ref):
  v = _select_sub(v_ref, vs_ref)
  r = _select_sub(r_ref, rs_ref)
  c = _select_sub(c_ref, cs_ref)
  feat = jnp.concatenate([v, p_ref[...], r, c], axis=-1)
  h = jnp.maximum(
      jnp.dot(feat, w0[...], preferred_element_type=jnp.float32) + b0[...], 0.0)
  h = jnp.maximum(
      jnp.dot(h, w1[...], preferred_element_type=jnp.float32) + b1[...], 0.0)
  h = jnp.maximum(
      jnp.dot(h, w2[...], preferred_element_type=jnp.float32) + b2[...], 0.0)
  out_ref[...] = jnp.dot(h, wo[...], preferred_element_type=jnp.float32) + bo[...]


def kernel(vid, watch_vids, region, cid, vemb, remb, cemb,
           W0, b0, W1, b1, W2, b2, Wo, bo):
  vid = vid.astype(jnp.int32)
  region = region.astype(jnp.int32)
  cid = cid.astype(jnp.int32)
  wvt = watch_vids.astype(jnp.int32).T
  vemb4 = vemb.reshape(-1, BLK)
  remb4 = remb.reshape(-1, BLK)
  cemb4 = cemb.reshape(-1, BLK)
  v4, pooled, r4, c4 = _sc_gather_pool(
      vid >> 2, wvt, region >> 2, cid >> 2, vemb4, remb4, cemb4)
  logit = pl.pallas_call(
      _mlp_body,
      out_shape=jax.ShapeDtypeStruct((B, 1), jnp.float32),
  )(v4, (vid & 3).reshape(B, 1), pooled,
    r4, (region & 3).reshape(B, 1), c4, (cid & 3).reshape(B, 1),
    W0, b0.reshape(1, -1), W1, b1.reshape(1, -1),
    W2, b2.reshape(1, -1), Wo, bo.reshape(1, -1))
  return logit
